# Initial kernel scaffold; baseline (speedup 1.0000x reference)
import jax
import jax.numpy as jnp
from jax.experimental import pallas as pl


def kernel(x, edge_index, W0, b0, W1, b1):
    raise NotImplementedError("write your pallas kernel here")


# speedup vs baseline: 7.9432x; 7.9432x over previous
import functools

import jax
import jax.numpy as jnp
from jax import lax
from jax.experimental import pallas as pl
from jax.experimental.pallas import tpu as pltpu
from jax.experimental.pallas import tpu_sc as plsc

N = 10000
E = 320000
IN_C = 128
HID = 64

NC, NS, L = 2, 16, 16
NW = NC * NS
CHUNK = 128
N_CHUNKS = -(-E // (NW * CHUNK))
E_TILE = N_CHUNKS * CHUNK
E_PAD = E_TILE * NW
N_PAD = 10240
DUMP = N
DEG_ROWS = N_PAD // NS

_mesh = plsc.VectorSubcoreMesh(core_axis_name="c", subcore_axis_name="s")


@functools.partial(
    pl.kernel,
    out_type=(
        jax.ShapeDtypeStruct((NC, NS, DEG_ROWS), jnp.float32),
        jax.ShapeDtypeStruct((E_PAD,), jnp.int32),
    ),
    mesh=_mesh,
    scratch_types=[
        pltpu.VMEM_SHARED((N_PAD,), jnp.float32),
        pltpu.VMEM((CHUNK,), jnp.int32),
        pltpu.VMEM((CHUNK,), jnp.int32),
        pltpu.VMEM((CHUNK,), jnp.int32),
        pltpu.VMEM((CHUNK,), jnp.float32),
        pltpu.VMEM((DEG_ROWS,), jnp.float32),
    ],
)
def _prep(src_hbm, dst_hbm, degp_hbm, dstm_hbm,
          deg_sh, src_v, dst_v, dstm_v, ones_v, degrow_v):
    c = lax.axis_index("c")
    s = lax.axis_index("s")
    wid = c * NS + s

    for j in range(CHUNK // L):
        ones_v[pl.ds(j * L, L)] = jnp.ones((L,), jnp.float32)

    def zero_row(i, carry):
        degrow_v[pl.ds(i * L, L)] = jnp.zeros((L,), jnp.float32)
        return carry

    lax.fori_loop(0, DEG_ROWS // L, zero_row, 0)
    pltpu.sync_copy(degrow_v, deg_sh.at[pl.ds(s * DEG_ROWS, DEG_ROWS)])
    plsc.subcore_barrier()

    def body(i, carry):
        ebase = wid * E_TILE + i * CHUNK
        pltpu.sync_copy(src_hbm.at[pl.ds(ebase, CHUNK)], src_v)
        pltpu.sync_copy(dst_hbm.at[pl.ds(ebase, CHUNK)], dst_v)
        for j in range(CHUNK // L):
            sv = src_v[pl.ds(j * L, L)]
            dv = dst_v[pl.ds(j * L, L)]
            dstm_v[pl.ds(j * L, L)] = jnp.where(sv == dv, DUMP, dv)
        pltpu.sync_copy(dstm_v, dstm_hbm.at[pl.ds(ebase, CHUNK)])
        pltpu.sync_copy(ones_v, deg_sh.at[dstm_v], add=True)
        return carry

    lax.fori_loop(0, N_CHUNKS, body, 0)
    plsc.subcore_barrier()

    pltpu.sync_copy(deg_sh.at[pl.ds(s * DEG_ROWS, DEG_ROWS)], degrow_v)
    pltpu.sync_copy(degrow_v, degp_hbm.at[c, s])


def _mlp_body(x_ref, w0_ref, b0_ref, w1_ref, b1_ref, degp_ref,
              g0_ref, dinv_ref, dinv2_ref):
    h = jnp.dot(x_ref[...], w0_ref[...], preferred_element_type=jnp.float32)
    h = h + b0_ref[...][None, :]
    mean = jnp.mean(h, axis=0, keepdims=True)
    var = jnp.mean(jnp.square(h - mean), axis=0, keepdims=True)
    h = (h - mean) * lax.rsqrt(var + 1e-10)
    h = jnp.maximum(h, 0.0)
    h = jnp.dot(h, w1_ref[...], preferred_element_type=jnp.float32)
    h = h + b1_ref[...][None, :]
    deg = jnp.sum(degp_ref[...], axis=1, keepdims=True) + 1.0
    dinv2 = 1.0 / deg[:N]
    dinv = lax.rsqrt(deg[:N])
    dinv_ref[...] = dinv
    dinv2_ref[...] = dinv2
    g0_ref[...] = h * dinv


_mlp = pl.pallas_call(
    _mlp_body,
    out_shape=(
        jax.ShapeDtypeStruct((N, IN_C), jnp.float32),
        jax.ShapeDtypeStruct((N, 1), jnp.float32),
        jax.ShapeDtypeStruct((N, 1), jnp.float32),
    ),
)


@functools.partial(
    pl.kernel,
    out_type=jax.ShapeDtypeStruct((NC, N_PAD, IN_C), jnp.float32),
    mesh=_mesh,
    scratch_types=[
        pltpu.VMEM_SHARED((N_PAD, IN_C), jnp.float32),
        pltpu.VMEM((CHUNK, IN_C), jnp.float32),
        pltpu.VMEM((CHUNK,), jnp.int32),
        pltpu.VMEM((CHUNK,), jnp.int32),
        pltpu.SemaphoreType.DMA,
    ],
)
def _round(src_hbm, dstm_hbm, g_hbm, p_hbm, acc_sh, rows_v, sidx_v, didx_v, sem):
    c = lax.axis_index("c")
    s = lax.axis_index("s")
    wid = c * NS + s

    def zero_row(i, carry):
        for j in range(IN_C // L):
            rows_v[i, pl.ds(j * L, L)] = jnp.zeros((L,), jnp.float32)
        return carry

    lax.fori_loop(0, CHUNK, zero_row, 0)
    for k in range(DEG_ROWS // CHUNK):
        pltpu.sync_copy(rows_v, acc_sh.at[pl.ds(s * DEG_ROWS + k * CHUNK, CHUNK)])
    plsc.subcore_barrier()

    def body(i, carry):
        ebase = wid * E_TILE + i * CHUNK
        pltpu.sync_copy(src_hbm.at[pl.ds(ebase, CHUNK)], sidx_v)
        pltpu.async_copy(g_hbm.at[sidx_v], rows_v, sem).wait()
        pltpu.sync_copy(dstm_hbm.at[pl.ds(ebase, CHUNK)], didx_v)
        pltpu.sync_copy(rows_v, acc_sh.at[didx_v], add=True)
        return carry

    lax.fori_loop(0, N_CHUNKS, body, 0)
    plsc.subcore_barrier()

    for k in range(DEG_ROWS // CHUNK):
        off = s * DEG_ROWS + k * CHUNK
        pltpu.sync_copy(acc_sh.at[pl.ds(off, CHUNK)], rows_v)
        pltpu.sync_copy(rows_v, p_hbm.at[c, pl.ds(off, CHUNK)])


def _combine_body(scale_ref, g_ref, p_ref, out_ref):
    p0 = p_ref[0, :N, :]
    p1 = p_ref[1, :N, :]
    out_ref[...] = scale_ref[...] * (g_ref[...] + p0 + p1)


_combine = pl.pallas_call(
    _combine_body,
    out_shape=jax.ShapeDtypeStruct((N, IN_C), jnp.float32),
)


def kernel(x, edge_index, W0, b0, W1, b1):
    src = edge_index[0]
    dst = edge_index[1]
    pad = E_PAD - E
    zpad = jnp.zeros((pad,), jnp.int32)
    src_p = jnp.concatenate([src, zpad])
    dst_p = jnp.concatenate([dst, zpad])

    degp, dstm = _prep(src_p, dst_p)
    degp_t = jnp.transpose(degp.reshape(NC, N_PAD), (1, 0))

    g, dinv, dinv2 = _mlp(x, W0, b0, W1, b1, degp_t)
    for r in range(5):
        p = _round(src_p, dstm, g)
        g = _combine(dinv2 if r < 4 else dinv, g, p)
    return g
